# baseline (speedup 1.0000x reference)
import jax
import jax.numpy as jnp
from jax.experimental import pallas as pl
from jax.experimental.pallas import tpu as pltpu

D_MODEL = 256
N_HEADS = 8
DH = D_MODEL // N_HEADS
TOP_K_N = 8
NEG_INF = float("-inf")
QB = 128
HPG = 8
N_G = N_HEADS // HPG


def _attn_pair_kernel(x_ref, wq_ref, wk_ref, wv_ref, wo_ref, out_ref,
                      *works):
    g = pl.program_id(0)
    T = x_ref.shape[0]
    n_qb = T // QB
    x = x_ref[...]
    qq = jax.lax.dot_general(x, wq_ref[...], (((1,), (1,)), ((), ())),
                             preferred_element_type=jnp.float32)
    kk = jax.lax.dot_general(x, wk_ref[...], (((1,), (1,)), ((), ())),
                             preferred_element_type=jnp.float32)
    vv = jax.lax.dot_general(x, wv_ref[...], (((1,), (1,)), ((), ())),
                             preferred_element_type=jnp.float32)
    kt_all = kk.T

    dr = jax.lax.broadcasted_iota(jnp.int32, (QB, QB), 0)
    dc = jax.lax.broadcasted_iota(jnp.int32, (QB, QB), 1)
    diag_mask = dc > dr

    colf = jax.lax.broadcasted_iota(jnp.int32, (T, T), 1).astype(jnp.float32)
    sent = float(T)

    for hh in range(HPG):
        q = qq[:, hh * DH:(hh + 1) * DH]
        kt = kt_all[hh * DH:(hh + 1) * DH, :]
        wref = works[hh]
        for qb in range(n_qb):
            W = (qb + 1) * QB
            qs = q[qb * QB:(qb + 1) * QB, :]
            sc = qs[:, 0:1] + kt[0:1, :W]
            for d in range(1, DH):
                sc = jnp.maximum(sc, qs[:, d:d + 1] + kt[d:d + 1, :W])
            if qb > 0:
                wref[qb * QB:(qb + 1) * QB, :qb * QB] = sc[:, :qb * QB]
            wref[qb * QB:(qb + 1) * QB, qb * QB:W] = jnp.where(
                diag_mask, NEG_INF, sc[:, qb * QB:W])
            if W < T:
                @pl.when(g == 0)
                def _fill():
                    wref[qb * QB:(qb + 1) * QB, W:] = jnp.full(
                        (QB, T - W), NEG_INF, jnp.float32)

    vals = {hh: [] for hh in range(HPG)}
    idxs = {hh: [] for hh in range(HPG)}
    for m in range(TOP_K_N):
        for hh in range(HPG):
            a = works[hh][...]
            vm = jnp.max(a, axis=1, keepdims=True)
            idx = jnp.min(jnp.where(a == vm, colf, sent), axis=1,
                          keepdims=True)
            if m < TOP_K_N - 1:
                works[hh][...] = jnp.where(colf == idx, NEG_INF, a)
            vals[hh].append(vm)
            idxs[hh].append(idx)

    contrib = None
    for hh in range(HPG):
        v0 = vals[hh][0]
        es = [jnp.exp(vm - v0) for vm in vals[hh]]
        denom = es[0]
        for e in es[1:]:
            denom = denom + e
        rden = 1.0 / denom

        v = vv[:, hh * DH:(hh + 1) * DH]
        ctx_blocks = []
        for qb in range(n_qb):
            W = (qb + 1) * QB
            r0, r1 = qb * QB, (qb + 1) * QB
            colb = colf[:QB, :W]
            num = jnp.where(colb == idxs[hh][0][r0:r1],
                            es[0][r0:r1], 0.0)
            for m in range(1, TOP_K_N):
                num = num + jnp.where(colb == idxs[hh][m][r0:r1],
                                      es[m][r0:r1], 0.0)
            ctx_blocks.append(jnp.dot(num * rden[r0:r1], v[:W, :],
                                      preferred_element_type=jnp.float32))
        ctx = jnp.concatenate(ctx_blocks, axis=0)
        c = jnp.dot(ctx, wo_ref[hh * DH:(hh + 1) * DH, :],
                    preferred_element_type=jnp.float32)
        contrib = c if contrib is None else contrib + c

    @pl.when(g == 0)
    def _init():
        out_ref[...] = contrib

    @pl.when(g != 0)
    def _acc():
        out_ref[...] += contrib


@jax.jit
def kernel(x, Wq, Wk, Wv, Wo):
    B, T, D = x.shape
    x2 = x.reshape(T, D)
    out = pl.pallas_call(
        _attn_pair_kernel,
        grid=(N_G,),
        in_specs=[
            pl.BlockSpec((T, D), lambda g: (0, 0)),
            pl.BlockSpec((HPG * DH, D), lambda g: (g, 0)),
            pl.BlockSpec((HPG * DH, D), lambda g: (g, 0)),
            pl.BlockSpec((HPG * DH, D), lambda g: (g, 0)),
            pl.BlockSpec((HPG * DH, D), lambda g: (g, 0)),
        ],
        out_specs=pl.BlockSpec((T, D), lambda g: (0, 0)),
        out_shape=jax.ShapeDtypeStruct((T, D), jnp.float32),
        scratch_shapes=[
            pltpu.VMEM((T, T), jnp.float32) for _ in range(HPG)
        ],
    )(x2, Wq, Wk, Wv, Wo.T)
    return out.reshape(B, T, D)
